# baseline (speedup 1.0000x reference)
import functools

import jax
import jax.numpy as jnp
from jax.experimental import pallas as pl
from jax.experimental.pallas import tpu as pltpu

H = 384
W = 384
HW = H * W
THING_LIST = [1, 2]
THRESHOLD = 0.1
NMS_KERNEL = 7
K_MAX = 4096

ROWS = 8
UNROLL = 32
NSPLIT = 12
CROWS = 32
CCOLS = K_MAX // CROWS


def _group_body(cy_ref, cx_ref, cyv_ref, cxv_ref, offy_ref, offx_ref, sem_ref,
                pan_ref, dist_ref):
    r = pl.program_id(0)
    row0 = (r * ROWS).astype(jnp.float32)
    iota_r = jax.lax.broadcasted_iota(jnp.int32, (ROWS, W), 0).astype(jnp.float32)
    iota_c = jax.lax.broadcasted_iota(jnp.int32, (ROWS, W), 1).astype(jnp.float32)
    ly = (row0 + iota_r) + offy_ref[...]
    lx = iota_c + offx_ref[...]

    cyv = cyv_ref[...]
    cxv = cxv_ref[...]
    ymin = jnp.min(ly)
    ymax = jnp.max(ly)
    dy_far = jnp.maximum(ymax - cyv, cyv - ymin)
    u2 = jnp.float32(0.0)
    wsplit = W // NSPLIT
    for k in range(NSPLIT):
        sl = lx[:, k * wsplit:(k + 1) * wsplit]
        xlo = jnp.min(sl)
        xhi = jnp.max(sl)
        dx_far = jnp.maximum(xhi - cxv, cxv - xlo)
        dmax2 = dy_far * dy_far + dx_far * dx_far
        u2 = jnp.maximum(u2, jnp.min(dmax2))
    u = jnp.sqrt(u2) + 1.0
    ylo = ymin - u
    yhi = ymax + u
    c_lo = jnp.sum((cyv < ylo).astype(jnp.int32))
    c_hi = jnp.sum((cyv <= yhi).astype(jnp.int32))

    def body(i, carry):
        b2, bi = carry
        c0 = i * UNROLL
        for u_ in range(UNROLL):
            c = c0 + u_
            cy = cy_ref[c]
            cx = cx_ref[c]
            dy = ly - cy
            dx = lx - cx
            d2 = dy * dy + dx * dx
            m = d2 < b2
            bi = jnp.where(m, c + 1, bi)
            b2 = jnp.minimum(b2, d2)
        return b2, bi

    b20 = jnp.full((ROWS, W), 1e10, jnp.float32)
    bi0 = jnp.zeros((ROWS, W), jnp.int32)
    b2, bi = jax.lax.fori_loop(c_lo // UNROLL, (c_hi + UNROLL - 1) // UNROLL,
                               body, (b20, bi0))

    sem = sem_ref[...]
    thing = (sem == THING_LIST[0]) | (sem == THING_LIST[1])
    pan_ref[...] = jnp.where(thing, bi, 0)
    dist_ref[...] = jnp.where(bi == 0, 1e5, jnp.sqrt(b2))


@jax.jit
def kernel(sem_seg, ctr_hmp, offsets):
    hmp = jnp.where(ctr_hmp > THRESHOLD, ctr_hmp, -1.0)
    pooled = jax.lax.reduce_window(hmp, -jnp.inf, jax.lax.max,
                                   (1, 1, NMS_KERNEL, NMS_KERNEL),
                                   (1, 1, 1, 1), 'SAME')
    keep = jnp.logical_and(hmp == pooled, hmp > 0.0)[0, 0]

    (idx,) = jnp.nonzero(keep.reshape(-1), size=K_MAX, fill_value=HW)
    valid = idx < HW
    idx_c = jnp.minimum(idx, HW - 1)
    cy = (idx_c // W).astype(jnp.float32)
    cx = (idx_c % W).astype(jnp.float32)
    ctr = jnp.stack([cy, cx], axis=-1)
    ctr = jnp.where(valid[:, None], ctr, 1e6)

    offy = offsets[0, 0]
    offx = offsets[0, 1]
    sem = sem_seg[0, 0]
    cy1 = ctr[:, 0]
    cx1 = ctr[:, 1]
    cyv = cy1.reshape(CROWS, CCOLS)
    cxv = cx1.reshape(CROWS, CCOLS)

    grid = (H // ROWS,)
    full = lambda r: (0, 0)
    tile = lambda r: (r, 0)
    pan, dist = pl.pallas_call(
        _group_body,
        grid=grid,
        in_specs=[
            pl.BlockSpec(memory_space=pltpu.SMEM),
            pl.BlockSpec(memory_space=pltpu.SMEM),
            pl.BlockSpec((CROWS, CCOLS), full),
            pl.BlockSpec((CROWS, CCOLS), full),
            pl.BlockSpec((ROWS, W), tile),
            pl.BlockSpec((ROWS, W), tile),
            pl.BlockSpec((ROWS, W), tile),
        ],
        out_specs=[
            pl.BlockSpec((ROWS, W), tile),
            pl.BlockSpec((ROWS, W), tile),
        ],
        out_shape=[
            jax.ShapeDtypeStruct((H, W), jnp.int32),
            jax.ShapeDtypeStruct((H, W), jnp.float32),
        ],
    )(cy1, cx1, cyv, cxv, offy, offx, sem)

    return pan[None], ctr[None], dist[None]


# speedup vs baseline: 1.2011x; 1.2011x over previous
import functools

import jax
import jax.numpy as jnp
from jax.experimental import pallas as pl
from jax.experimental.pallas import tpu as pltpu

H = 384
W = 384
HW = H * W
THING_LIST = [1, 2]
THRESHOLD = 0.1
NMS_KERNEL = 7
K_MAX = 4096

ROWS = 16
UNROLL = 16
NSPLIT = 12
CROWS = 32
CCOLS = K_MAX // CROWS


def _group_body(cy_ref, cx_ref, cyv_ref, cxv_ref, offy_ref, offx_ref, sem_ref,
                pan_ref, dist_ref):
    r = pl.program_id(0)
    row0 = (r * ROWS).astype(jnp.float32)
    iota_r = jax.lax.broadcasted_iota(jnp.int32, (ROWS, W), 0).astype(jnp.float32)
    iota_c = jax.lax.broadcasted_iota(jnp.int32, (ROWS, W), 1).astype(jnp.float32)
    ly = (row0 + iota_r) + offy_ref[...]
    lx = iota_c + offx_ref[...]

    cyv = cyv_ref[...]
    cxv = cxv_ref[...]
    ymin = jnp.min(ly)
    ymax = jnp.max(ly)
    dy_far = jnp.maximum(ymax - cyv, cyv - ymin)
    u2 = jnp.float32(0.0)
    wsplit = W // NSPLIT
    for k in range(NSPLIT):
        sl = lx[:, k * wsplit:(k + 1) * wsplit]
        xlo = jnp.min(sl)
        xhi = jnp.max(sl)
        dx_far = jnp.maximum(xhi - cxv, cxv - xlo)
        dmax2 = dy_far * dy_far + dx_far * dx_far
        u2 = jnp.maximum(u2, jnp.min(dmax2))
    u = jnp.sqrt(u2) + 1.0
    ylo = ymin - u
    yhi = ymax + u
    c_lo = jnp.sum((cyv < ylo).astype(jnp.int32))
    c_hi = jnp.sum((cyv <= yhi).astype(jnp.int32))

    def body(i, carry):
        b2, bi = carry
        c0 = i * UNROLL
        for u_ in range(UNROLL):
            c = c0 + u_
            cy = cy_ref[c]
            cx = cx_ref[c]
            dy = ly - cy
            dx = lx - cx
            d2 = dy * dy + dx * dx
            m = d2 < b2
            bi = jnp.where(m, c + 1, bi)
            b2 = jnp.minimum(b2, d2)
        return b2, bi

    b20 = jnp.full((ROWS, W), 1e10, jnp.float32)
    bi0 = jnp.zeros((ROWS, W), jnp.int32)
    b2, bi = jax.lax.fori_loop(c_lo // UNROLL, (c_hi + UNROLL - 1) // UNROLL,
                               body, (b20, bi0))

    sem = sem_ref[...]
    thing = (sem == THING_LIST[0]) | (sem == THING_LIST[1])
    pan_ref[...] = jnp.where(thing, bi, 0)
    dist_ref[...] = jnp.where(bi == 0, 1e5, jnp.sqrt(b2))


@jax.jit
def kernel(sem_seg, ctr_hmp, offsets):
    hmp = jnp.where(ctr_hmp > THRESHOLD, ctr_hmp, -1.0)
    pooled = jax.lax.reduce_window(hmp, -jnp.inf, jax.lax.max,
                                   (1, 1, NMS_KERNEL, NMS_KERNEL),
                                   (1, 1, 1, 1), 'SAME')
    keep = jnp.logical_and(hmp == pooled, hmp > 0.0)[0, 0]

    (idx,) = jnp.nonzero(keep.reshape(-1), size=K_MAX, fill_value=HW)
    valid = idx < HW
    idx_c = jnp.minimum(idx, HW - 1)
    cy = (idx_c // W).astype(jnp.float32)
    cx = (idx_c % W).astype(jnp.float32)
    ctr = jnp.stack([cy, cx], axis=-1)
    ctr = jnp.where(valid[:, None], ctr, 1e6)

    offy = offsets[0, 0]
    offx = offsets[0, 1]
    sem = sem_seg[0, 0]
    cy1 = ctr[:, 0]
    cx1 = ctr[:, 1]
    cyv = cy1.reshape(CROWS, CCOLS)
    cxv = cx1.reshape(CROWS, CCOLS)

    grid = (H // ROWS,)
    full = lambda r: (0, 0)
    tile = lambda r: (r, 0)
    pan, dist = pl.pallas_call(
        _group_body,
        grid=grid,
        in_specs=[
            pl.BlockSpec(memory_space=pltpu.SMEM),
            pl.BlockSpec(memory_space=pltpu.SMEM),
            pl.BlockSpec((CROWS, CCOLS), full),
            pl.BlockSpec((CROWS, CCOLS), full),
            pl.BlockSpec((ROWS, W), tile),
            pl.BlockSpec((ROWS, W), tile),
            pl.BlockSpec((ROWS, W), tile),
        ],
        out_specs=[
            pl.BlockSpec((ROWS, W), tile),
            pl.BlockSpec((ROWS, W), tile),
        ],
        out_shape=[
            jax.ShapeDtypeStruct((H, W), jnp.int32),
            jax.ShapeDtypeStruct((H, W), jnp.float32),
        ],
    )(cy1, cx1, cyv, cxv, offy, offx, sem)

    return pan[None], ctr[None], dist[None]
